# baseline (speedup 1.0000x reference)
import functools

import jax
import jax.numpy as jnp
from jax import lax
from jax.experimental import pallas as pl
from jax.experimental.pallas import tpu as pltpu
from jax.experimental.pallas import tpu_sc as plsc

N = 10000
E = 160000
D_IN = 128
D_HID = 128
D_OUT = 64
K = 16

NC = 2
NS = 16
NW = NC * NS

NPAD = 10240
ROWS_PER_SUB = NPAD // NS
STG = ROWS_PER_SUB // 2

EPAD = 172032
SLOW_CORE = 0
CHUNK1 = 64
NCHF1, NCHS1 = 111, 57
CHUNK2 = 128
NCHF2, NCHS2 = 54, 30

DP1 = 144
DP2 = 80

_mesh = plsc.VectorSubcoreMesh(
    core_axis_name="c", subcore_axis_name="s", num_cores=NC, num_subcores=NS
)


def _make_sc_gat(dp):
    acol = dp - 15

    def make_sc(chunk, nch_fast, nch_slow, slow_core):
        assert nch_fast % 3 == 0 and nch_slow % 3 == 0
        peel = 3

        def body(hp, adt, src3, dst3, zrows, out,
                 sidx0, sidx1, sidx2, didx0, didx1, didx2,
                 dsc0, dsc1, dsc2, wc_v, rows0, rows1, rows2,
                 alr0, alr1, alr2,
                 si0, si1, si2, sg0, sg1, sg2, ss0, ss1, ss2, acc_sh):
            c = lax.axis_index("c")
            s = lax.axis_index("s")
            wid = s * NC + c
            nchunk = jnp.where(c == slow_core, nch_slow, nch_fast)
            last = nchunk - 1
            nloops = nchunk // 3 - 1
            sidx = (sidx0, sidx1, sidx2)
            didx = (didx0, didx1, didx2)
            dsc = (dsc0, dsc1, dsc2)
            rows = (rows0, rows1, rows2)
            alr = (alr0, alr1, alr2)
            sem_i = (si0, si1, si2)
            sem_g = (sg0, sg1, sg2)
            sem_s = (ss0, ss1, ss2)

            def issue_idx(ci, b):
                pltpu.async_copy(src3.at[wid, ci], sidx[b], sem_i[b])
                pltpu.async_copy(dst3.at[wid, ci], didx[b], sem_i[b])

            def wait_idx(b):
                pltpu.make_async_copy(src3.at[wid, 0], sidx[b], sem_i[b]).wait()
                pltpu.make_async_copy(dst3.at[wid, 0], didx[b], sem_i[b]).wait()

            def issue_gather(b):
                pltpu.async_copy(hp.at[sidx[b]], rows[b], sem_g[b])
                pltpu.async_copy(adt.at[didx[b]], alr[b], sem_g[b])

            def wait_gather(b):
                pltpu.make_async_copy(hp.at[sidx[b]], rows[b], sem_g[b]).wait()
                pltpu.make_async_copy(adt.at[didx[b]], alr[b], sem_g[b]).wait()

            def wait_scat(b):
                pltpu.make_async_copy(rows[b], acc_sh.at[dsc[b]],
                                      sem_s[b]).wait()

            issue_idx(0, 0)
            issue_idx(1, 1)
            issue_idx(2, 2)

            pltpu.sync_copy(zrows, rows0)
            base = s * ROWS_PER_SUB
            nzc = ROWS_PER_SUB // chunk
            for k in range(nzc):
                pltpu.sync_copy(rows0, acc_sh.at[pl.ds(base + k * chunk, chunk)])
            rem = ROWS_PER_SUB - nzc * chunk
            if rem:
                pltpu.sync_copy(rows0.at[pl.ds(0, rem)],
                                acc_sh.at[pl.ds(base + nzc * chunk, rem)])
            plsc.subcore_barrier()

            wait_idx(0)
            issue_gather(0)

            iota16 = lax.iota(jnp.int32, 16)
            zeros16 = jnp.zeros((16,), jnp.int32)
            acol16 = jnp.full((16,), acol, jnp.int32)

            def phase(ci, b, first):
                p = (b + 1) % 3
                if not first:
                    wait_scat(p)
                wait_idx(p)
                issue_gather(p)
                wait_gather(b)
                for j in range(chunk // 16):
                    sl = pl.ds(j * 16, 16)
                    dsc[b][sl] = didx[b][sl]
                issue_idx(jnp.minimum(ci + 3, last), b)
                for j in range(chunk // 16):
                    lane = j * 16 + iota16
                    av = plsc.load_gather(rows[b], [lane, acol16])
                    dv = plsc.load_gather(alr[b], [lane, zeros16])
                    e = av + dv
                    e = jnp.where(e >= 0.0, e, 0.2 * e)
                    wc_v[pl.ds(j * 16, 16)] = jnp.exp(e)
                def scale_step(ei, _):
                    wsp = plsc.load_gather(
                        wc_v, [jnp.full((16,), ei, jnp.int32)])
                    for j in range(dp // 16):
                        sl = pl.ds(j * 16, 16)
                        rows[b][ei, sl] = rows[b][ei, sl] * wsp
                    return 0
                lax.fori_loop(0, chunk, scale_step, 0, unroll=2)
                pltpu.async_copy(rows[b], acc_sh.at[dsc[b]], sem_s[b],
                                 add=True)

            for ci in range(peel):
                phase(ci, ci % 3, ci < 2)

            def loop_body(i3, _):
                ci = peel + 3 * i3
                phase(ci, peel % 3, False)
                phase(ci + 1, (peel + 1) % 3, False)
                phase(ci + 2, (peel + 2) % 3, False)
                return 0

            lax.fori_loop(0, nloops, loop_body, 0)

            wait_scat(1)
            wait_scat(2)
            wait_idx(1)
            wait_idx(2)
            wait_gather(0)
            plsc.subcore_barrier()

            for k in range(nzc):
                sl = pl.ds(base + k * chunk, chunk)
                pltpu.sync_copy(acc_sh.at[sl], rows0)
                pltpu.sync_copy(rows0, out.at[c, sl])
            if rem:
                sl = pl.ds(base + nzc * chunk, rem)
                pltpu.sync_copy(acc_sh.at[sl], rows0.at[pl.ds(0, rem)])
                pltpu.sync_copy(rows0.at[pl.ds(0, rem)], out.at[c, sl])

        return pl.kernel(
            body,
            out_type=jax.ShapeDtypeStruct((NC, NPAD, dp), jnp.float32),
            mesh=_mesh,
            compiler_params=pltpu.CompilerParams(
                needs_layout_passes=False, use_tc_tiling_on_sc=False),
            scratch_types=(
                [pltpu.VMEM((chunk,), jnp.int32)] * 6 +
                [pltpu.VMEM((chunk,), jnp.int32)] * 3 +
                [pltpu.VMEM((chunk,), jnp.float32)] +
                [pltpu.VMEM((chunk, dp), jnp.float32)] * 3 +
                [pltpu.VMEM((chunk, 16), jnp.float32)] * 3 +
                [pltpu.SemaphoreType.DMA] * 9 +
                [pltpu.VMEM_SHARED((NPAD, dp), jnp.float32)]
            ),
        )

    return make_sc


_sc_gat1 = _make_sc_gat(DP1)(CHUNK1, NCHF1, NCHS1, SLOW_CORE)
_sc_gat2 = _make_sc_gat(DP2)(CHUNK2, NCHF2, NCHS2, SLOW_CORE)


def _edge_layout(flat, chunk, nf, ns):
    lf, ls = nf * chunk, ns * chunk
    seg_f = flat[:NS * lf].reshape(NS, lf)
    seg_s = flat[NS * lf:NS * (lf + ls)].reshape(NS, ls)
    arr = jnp.full((NW, nf * chunk), N, jnp.int32)
    arr = arr.at[(1 - SLOW_CORE)::2, :lf].set(seg_f)
    arr = arr.at[SLOW_CORE::2, :ls].set(seg_s)
    return arr.reshape(NW, nf, chunk)


BLK_A = 1280


def _tail_cols(blk, al, nblk):
    rows = pl.program_id(0) * nblk + lax.broadcasted_iota(
        jnp.int32, (nblk, 16), 0)
    lanes = lax.broadcasted_iota(jnp.int32, (nblk, 16), 1)
    ones = jnp.where((lanes == 0) & (rows < N), 1.0, 0.0)
    return ones + jnp.where(lanes == 1, al, 0.0)


def _pre1_body(x_ref, w_ref, a_ref, hp_ref, adt_ref):
    xb = x_ref[...]
    h = jax.lax.dot_general(xb, w_ref[...], (((1,), (0,)), ((), ())),
                            preferred_element_type=jnp.float32)
    al = jax.lax.dot_general(h, a_ref[...], (((1,), (1,)), ((), ())),
                             preferred_element_type=jnp.float32)
    hp_ref[:, :D_HID] = h
    hp_ref[:, D_HID:DP1] = _tail_cols(BLK_A, al[:, 0:1], BLK_A)
    lanes = lax.broadcasted_iota(jnp.int32, (BLK_A, 16), 1)
    adt_ref[...] = jnp.where(lanes == 0, al[:, 1:2], 0.0)


def _pre1(x_pad, w1, a1):
    return pl.pallas_call(
        _pre1_body,
        grid=(NPAD // BLK_A,),
        in_specs=[
            pl.BlockSpec((BLK_A, D_IN), lambda i: (i, 0)),
            pl.BlockSpec((D_IN, D_HID), lambda i: (0, 0)),
            pl.BlockSpec((2, D_HID), lambda i: (0, 0)),
        ],
        out_specs=[
            pl.BlockSpec((BLK_A, DP1), lambda i: (i, 0)),
            pl.BlockSpec((BLK_A, 16), lambda i: (i, 0)),
        ],
        out_shape=[
            jax.ShapeDtypeStruct((NPAD, DP1), jnp.float32),
            jax.ShapeDtypeStruct((NPAD, 16), jnp.float32),
        ],
    )(x_pad, w1, a1)


BLK_B = 1280


def _mid_body(acc_ref, b1_ref, w2_ref, a2_ref, hp2_ref, adt2_ref):
    accs = acc_ref[0] + acc_ref[1]
    num = accs[:, :D_HID]
    den = accs[:, D_HID:D_HID + 1]
    h1 = num / jnp.maximum(den, 1e-30) + b1_ref[...]
    h1 = jnp.where(h1 > 0.0, h1, jnp.exp(jnp.minimum(h1, 0.0)) - 1.0)
    h2 = jax.lax.dot_general(h1, w2_ref[...], (((1,), (0,)), ((), ())),
                             preferred_element_type=jnp.float32)
    al = jax.lax.dot_general(h2, a2_ref[...], (((1,), (1,)), ((), ())),
                             preferred_element_type=jnp.float32)
    hp2_ref[:, :D_OUT] = h2
    hp2_ref[:, D_OUT:DP2] = _tail_cols(BLK_B, al[:, 0:1], BLK_B)
    lanes = lax.broadcasted_iota(jnp.int32, (BLK_B, 16), 1)
    adt2_ref[...] = jnp.where(lanes == 0, al[:, 1:2], 0.0)


def _mid(acc1, b1, w2, a2):
    return pl.pallas_call(
        _mid_body,
        grid=(NPAD // BLK_B,),
        in_specs=[
            pl.BlockSpec((NC, BLK_B, DP1), lambda i: (0, i, 0)),
            pl.BlockSpec((1, D_HID), lambda i: (0, 0)),
            pl.BlockSpec((D_HID, D_OUT), lambda i: (0, 0)),
            pl.BlockSpec((2, D_OUT), lambda i: (0, 0)),
        ],
        out_specs=[
            pl.BlockSpec((BLK_B, DP2), lambda i: (i, 0)),
            pl.BlockSpec((BLK_B, 16), lambda i: (i, 0)),
        ],
        out_shape=[
            jax.ShapeDtypeStruct((NPAD, DP2), jnp.float32),
            jax.ShapeDtypeStruct((NPAD, 16), jnp.float32),
        ],
    )(acc1, b1, w2, a2)


BLK_C = 2000


def _fin_body(acc_ref, b2_ref, z_ref):
    accs = acc_ref[0] + acc_ref[1]
    num = accs[:, :D_OUT]
    den = accs[:, D_OUT:D_OUT + 1]
    z_ref[...] = num / jnp.maximum(den, 1e-30) + b2_ref[...]


def _fin(acc2, b2):
    return pl.pallas_call(
        _fin_body,
        grid=(N // BLK_C,),
        in_specs=[
            pl.BlockSpec((NC, BLK_C, DP2), lambda i: (0, i, 0)),
            pl.BlockSpec((1, D_OUT), lambda i: (0, 0)),
        ],
        out_specs=pl.BlockSpec((BLK_C, D_OUT), lambda i: (i, 0)),
        out_shape=jax.ShapeDtypeStruct((N, D_OUT), jnp.float32),
    )(acc2, b2)


ROW_BLK = 400


def _dense_body(z_blk_ref, z_all_ref, cc_ref, a_ref, q_ref):
    zi = z_blk_ref[...]
    zall = z_all_ref[...]
    cc = cc_ref[...]
    sim = jax.lax.dot_general(zi, zall, (((1,), (1,)), ((), ())),
                              preferred_element_type=jnp.float32)
    a_ref[...] = jax.nn.sigmoid(sim)
    zc = jax.lax.dot_general(zi, cc, (((1,), (1,)), ((), ())),
                             preferred_element_type=jnp.float32)
    z2 = jnp.sum(zi * zi, axis=1, keepdims=True)
    c2 = jnp.sum(cc * cc, axis=1)[None, :]
    d2 = z2 - 2.0 * zc + c2
    qu = 1.0 / (1.0 + d2)
    q_ref[...] = qu / jnp.sum(qu, axis=1, keepdims=True)


def _dense_outputs(z, cluster_centers):
    return pl.pallas_call(
        _dense_body,
        grid=(N // ROW_BLK,),
        in_specs=[
            pl.BlockSpec((ROW_BLK, D_OUT), lambda i: (i, 0)),
            pl.BlockSpec((N, D_OUT), lambda i: (0, 0)),
            pl.BlockSpec((K, D_OUT), lambda i: (0, 0)),
        ],
        out_specs=[
            pl.BlockSpec((ROW_BLK, N), lambda i: (i, 0)),
            pl.BlockSpec((ROW_BLK, K), lambda i: (i, 0)),
        ],
        out_shape=[
            jax.ShapeDtypeStruct((N, N), jnp.float32),
            jax.ShapeDtypeStruct((N, K), jnp.float32),
        ],
    )(z, z, cluster_centers)


def kernel(x, edge_index, W1, a1_src, a1_dst, b1, W2, a2_src, a2_dst, b2,
           cluster_centers):
    src, dst = edge_index[0], edge_index[1]
    loop = jnp.arange(N, dtype=jnp.int32)
    fill = jnp.full((EPAD - E - N,), N, jnp.int32)
    src_flat = jnp.concatenate([src, loop, fill])
    dst_flat = jnp.concatenate([dst, loop, fill])
    src3a = _edge_layout(src_flat, CHUNK1, NCHF1, NCHS1)
    dst3a = _edge_layout(dst_flat, CHUNK1, NCHF1, NCHS1)
    src3b = _edge_layout(src_flat, CHUNK2, NCHF2, NCHS2)
    dst3b = _edge_layout(dst_flat, CHUNK2, NCHF2, NCHS2)

    x_pad = jnp.pad(x, ((0, NPAD - N), (0, 0)))
    a1 = jnp.stack([a1_src, a1_dst])
    a2 = jnp.stack([a2_src, a2_dst])
    zrows1 = jnp.zeros((CHUNK1, DP1), jnp.float32)
    zrows2 = jnp.zeros((CHUNK2, DP2), jnp.float32)

    hp1, adt1 = _pre1(x_pad, W1, a1)
    acc1 = _sc_gat1(hp1, adt1, src3a, dst3a, zrows1)
    hp2, adt2 = _mid(acc1, b1[None, :], W2, a2)
    acc2 = _sc_gat2(hp2, adt2, src3b, dst3b, zrows2)
    z = _fin(acc2, b2[None, :])
    a_pred, q = _dense_outputs(z, cluster_centers)
    return (z, a_pred, q)


# speedup vs baseline: 1.2802x; 1.2802x over previous
import functools

import jax
import jax.numpy as jnp
from jax import lax
from jax.experimental import pallas as pl
from jax.experimental.pallas import tpu as pltpu
from jax.experimental.pallas import tpu_sc as plsc

N = 10000
E = 160000
D_IN = 128
D_HID = 128
D_OUT = 64
K = 16

NC = 2
NS = 16
NW = NC * NS

NPAD = 10240
ROWS_PER_SUB = NPAD // NS
STG = ROWS_PER_SUB // 2

EPAD = 172032
SLOW_CORE = 0
CHUNK1 = 64
NCHF1, NCHS1 = 84, 84
CHUNK2 = 128
NCHF2, NCHS2 = 42, 42

DP1 = 144
DP2 = 80

_mesh = plsc.VectorSubcoreMesh(
    core_axis_name="c", subcore_axis_name="s", num_cores=NC, num_subcores=NS
)


def _make_sc_gat(dp):
    acol = dp - 15

    def make_sc(chunk, nch_fast, nch_slow, slow_core):
        assert nch_fast % 3 == 0 and nch_slow % 3 == 0
        peel = 3

        def body(hp, adt, src3, dst3, zrows, out,
                 sidx0, sidx1, sidx2, didx0, didx1, didx2,
                 dsc0, dsc1, dsc2, wc_v, rows0, rows1, rows2,
                 alr0, alr1, alr2,
                 si0, si1, si2, sg0, sg1, sg2, ss0, ss1, ss2, acc_sh):
            c = lax.axis_index("c")
            s = lax.axis_index("s")
            wid = s * NC + c
            nchunk = jnp.where(c == slow_core, nch_slow, nch_fast)
            last = nchunk - 1
            nloops = nchunk // 3 - 1
            sidx = (sidx0, sidx1, sidx2)
            didx = (didx0, didx1, didx2)
            dsc = (dsc0, dsc1, dsc2)
            rows = (rows0, rows1, rows2)
            alr = (alr0, alr1, alr2)
            sem_i = (si0, si1, si2)
            sem_g = (sg0, sg1, sg2)
            sem_s = (ss0, ss1, ss2)

            def issue_idx(ci, b):
                pltpu.async_copy(src3.at[wid, ci], sidx[b], sem_i[b])
                pltpu.async_copy(dst3.at[wid, ci], didx[b], sem_i[b])

            def wait_idx(b):
                pltpu.make_async_copy(src3.at[wid, 0], sidx[b], sem_i[b]).wait()
                pltpu.make_async_copy(dst3.at[wid, 0], didx[b], sem_i[b]).wait()

            def issue_gather(b):
                pltpu.async_copy(hp.at[sidx[b]], rows[b], sem_g[b])
                pltpu.async_copy(adt.at[didx[b]], alr[b], sem_g[b])

            def wait_gather(b):
                pltpu.make_async_copy(hp.at[sidx[b]], rows[b], sem_g[b]).wait()
                pltpu.make_async_copy(adt.at[didx[b]], alr[b], sem_g[b]).wait()

            def wait_scat(b):
                pltpu.make_async_copy(rows[b], acc_sh.at[dsc[b]],
                                      sem_s[b]).wait()

            issue_idx(0, 0)
            issue_idx(1, 1)
            issue_idx(2, 2)

            pltpu.sync_copy(zrows, rows0)
            base = s * ROWS_PER_SUB
            nzc = ROWS_PER_SUB // chunk
            for k in range(nzc):
                pltpu.sync_copy(rows0, acc_sh.at[pl.ds(base + k * chunk, chunk)])
            rem = ROWS_PER_SUB - nzc * chunk
            if rem:
                pltpu.sync_copy(rows0.at[pl.ds(0, rem)],
                                acc_sh.at[pl.ds(base + nzc * chunk, rem)])
            plsc.subcore_barrier()

            wait_idx(0)
            issue_gather(0)

            iota16 = lax.iota(jnp.int32, 16)
            zeros16 = jnp.zeros((16,), jnp.int32)
            acol16 = jnp.full((16,), acol, jnp.int32)

            def phase(ci, b, first):
                p = (b + 1) % 3
                if not first:
                    wait_scat(p)
                wait_idx(p)
                issue_gather(p)
                wait_gather(b)
                for j in range(chunk // 16):
                    sl = pl.ds(j * 16, 16)
                    dsc[b][sl] = didx[b][sl]
                issue_idx(jnp.minimum(ci + 3, last), b)
                for j in range(chunk // 16):
                    lane = j * 16 + iota16
                    av = plsc.load_gather(rows[b], [lane, acol16])
                    dv = plsc.load_gather(alr[b], [lane, zeros16])
                    e = av + dv
                    e = jnp.where(e >= 0.0, e, 0.2 * e)
                    wc_v[pl.ds(j * 16, 16)] = jnp.exp(e)
                def scale_step(ei, _):
                    wsp = plsc.load_gather(
                        wc_v, [jnp.full((16,), ei, jnp.int32)])
                    for j in range(dp // 16):
                        sl = pl.ds(j * 16, 16)
                        rows[b][ei, sl] = rows[b][ei, sl] * wsp
                    return 0
                lax.fori_loop(0, chunk, scale_step, 0, unroll=4)
                pltpu.async_copy(rows[b], acc_sh.at[dsc[b]], sem_s[b],
                                 add=True)

            for ci in range(peel):
                phase(ci, ci % 3, ci < 2)

            def loop_body(i3, _):
                ci = peel + 3 * i3
                phase(ci, peel % 3, False)
                phase(ci + 1, (peel + 1) % 3, False)
                phase(ci + 2, (peel + 2) % 3, False)
                return 0

            lax.fori_loop(0, nloops, loop_body, 0)

            wait_scat(1)
            wait_scat(2)
            wait_idx(1)
            wait_idx(2)
            wait_gather(0)
            plsc.subcore_barrier()

            for k in range(nzc):
                sl = pl.ds(base + k * chunk, chunk)
                pltpu.sync_copy(acc_sh.at[sl], rows0)
                pltpu.sync_copy(rows0, out.at[c, sl])
            if rem:
                sl = pl.ds(base + nzc * chunk, rem)
                pltpu.sync_copy(acc_sh.at[sl], rows0.at[pl.ds(0, rem)])
                pltpu.sync_copy(rows0.at[pl.ds(0, rem)], out.at[c, sl])

        return pl.kernel(
            body,
            out_type=jax.ShapeDtypeStruct((NC, NPAD, dp), jnp.float32),
            mesh=_mesh,
            compiler_params=pltpu.CompilerParams(
                needs_layout_passes=False, use_tc_tiling_on_sc=False),
            scratch_types=(
                [pltpu.VMEM((chunk,), jnp.int32)] * 6 +
                [pltpu.VMEM((chunk,), jnp.int32)] * 3 +
                [pltpu.VMEM((chunk,), jnp.float32)] +
                [pltpu.VMEM((chunk, dp), jnp.float32)] * 3 +
                [pltpu.VMEM((chunk, 16), jnp.float32)] * 3 +
                [pltpu.SemaphoreType.DMA] * 9 +
                [pltpu.VMEM_SHARED((NPAD, dp), jnp.float32)]
            ),
        )

    return make_sc


_sc_gat1 = _make_sc_gat(DP1)(CHUNK1, NCHF1, NCHS1, SLOW_CORE)
_sc_gat2 = _make_sc_gat(DP2)(CHUNK2, NCHF2, NCHS2, SLOW_CORE)


def _edge_layout(flat, chunk, nf, ns):
    if nf == ns:
        return flat.reshape(NW, nf, chunk)
    lf, ls = nf * chunk, ns * chunk
    seg_f = flat[:NS * lf].reshape(NS, lf)
    seg_s = flat[NS * lf:NS * (lf + ls)].reshape(NS, ls)
    arr = jnp.full((NW, nf * chunk), N, jnp.int32)
    arr = arr.at[(1 - SLOW_CORE)::2, :lf].set(seg_f)
    arr = arr.at[SLOW_CORE::2, :ls].set(seg_s)
    return arr.reshape(NW, nf, chunk)


BLK_A = 1280


def _tail_cols(blk, al, nblk):
    rows = pl.program_id(0) * nblk + lax.broadcasted_iota(
        jnp.int32, (nblk, 16), 0)
    lanes = lax.broadcasted_iota(jnp.int32, (nblk, 16), 1)
    ones = jnp.where((lanes == 0) & (rows < N), 1.0, 0.0)
    return ones + jnp.where(lanes == 1, al, 0.0)


def _pre1_body(x_ref, w_ref, a_ref, hp_ref, adt_ref):
    xb = x_ref[...]
    h = jax.lax.dot_general(xb, w_ref[...], (((1,), (0,)), ((), ())),
                            preferred_element_type=jnp.float32)
    al = jax.lax.dot_general(h, a_ref[...], (((1,), (1,)), ((), ())),
                             preferred_element_type=jnp.float32)
    hp_ref[:, :D_HID] = h
    hp_ref[:, D_HID:DP1] = _tail_cols(BLK_A, al[:, 0:1], BLK_A)
    lanes = lax.broadcasted_iota(jnp.int32, (BLK_A, 16), 1)
    adt_ref[...] = jnp.where(lanes == 0, al[:, 1:2], 0.0)


def _pre1(x_pad, w1, a1):
    return pl.pallas_call(
        _pre1_body,
        grid=(NPAD // BLK_A,),
        in_specs=[
            pl.BlockSpec((BLK_A, D_IN), lambda i: (i, 0)),
            pl.BlockSpec((D_IN, D_HID), lambda i: (0, 0)),
            pl.BlockSpec((2, D_HID), lambda i: (0, 0)),
        ],
        out_specs=[
            pl.BlockSpec((BLK_A, DP1), lambda i: (i, 0)),
            pl.BlockSpec((BLK_A, 16), lambda i: (i, 0)),
        ],
        out_shape=[
            jax.ShapeDtypeStruct((NPAD, DP1), jnp.float32),
            jax.ShapeDtypeStruct((NPAD, 16), jnp.float32),
        ],
    )(x_pad, w1, a1)


BLK_B = 1280


def _mid_body(acc_ref, b1_ref, w2_ref, a2_ref, hp2_ref, adt2_ref):
    accs = acc_ref[0] + acc_ref[1]
    num = accs[:, :D_HID]
    den = accs[:, D_HID:D_HID + 1]
    h1 = num / jnp.maximum(den, 1e-30) + b1_ref[...]
    h1 = jnp.where(h1 > 0.0, h1, jnp.exp(jnp.minimum(h1, 0.0)) - 1.0)
    h2 = jax.lax.dot_general(h1, w2_ref[...], (((1,), (0,)), ((), ())),
                             preferred_element_type=jnp.float32)
    al = jax.lax.dot_general(h2, a2_ref[...], (((1,), (1,)), ((), ())),
                             preferred_element_type=jnp.float32)
    hp2_ref[:, :D_OUT] = h2
    hp2_ref[:, D_OUT:DP2] = _tail_cols(BLK_B, al[:, 0:1], BLK_B)
    lanes = lax.broadcasted_iota(jnp.int32, (BLK_B, 16), 1)
    adt2_ref[...] = jnp.where(lanes == 0, al[:, 1:2], 0.0)


def _mid(acc1, b1, w2, a2):
    return pl.pallas_call(
        _mid_body,
        grid=(NPAD // BLK_B,),
        in_specs=[
            pl.BlockSpec((NC, BLK_B, DP1), lambda i: (0, i, 0)),
            pl.BlockSpec((1, D_HID), lambda i: (0, 0)),
            pl.BlockSpec((D_HID, D_OUT), lambda i: (0, 0)),
            pl.BlockSpec((2, D_OUT), lambda i: (0, 0)),
        ],
        out_specs=[
            pl.BlockSpec((BLK_B, DP2), lambda i: (i, 0)),
            pl.BlockSpec((BLK_B, 16), lambda i: (i, 0)),
        ],
        out_shape=[
            jax.ShapeDtypeStruct((NPAD, DP2), jnp.float32),
            jax.ShapeDtypeStruct((NPAD, 16), jnp.float32),
        ],
    )(acc1, b1, w2, a2)


BLK_C = 2000


def _fin_body(acc_ref, b2_ref, z_ref):
    accs = acc_ref[0] + acc_ref[1]
    num = accs[:, :D_OUT]
    den = accs[:, D_OUT:D_OUT + 1]
    z_ref[...] = num / jnp.maximum(den, 1e-30) + b2_ref[...]


def _fin(acc2, b2):
    return pl.pallas_call(
        _fin_body,
        grid=(N // BLK_C,),
        in_specs=[
            pl.BlockSpec((NC, BLK_C, DP2), lambda i: (0, i, 0)),
            pl.BlockSpec((1, D_OUT), lambda i: (0, 0)),
        ],
        out_specs=pl.BlockSpec((BLK_C, D_OUT), lambda i: (i, 0)),
        out_shape=jax.ShapeDtypeStruct((N, D_OUT), jnp.float32),
    )(acc2, b2)


ROW_BLK = 400


def _dense_body(z_blk_ref, z_all_ref, cc_ref, a_ref, q_ref):
    zi = z_blk_ref[...]
    zall = z_all_ref[...]
    cc = cc_ref[...]
    sim = jax.lax.dot_general(zi, zall, (((1,), (1,)), ((), ())),
                              preferred_element_type=jnp.float32)
    a_ref[...] = jax.nn.sigmoid(sim)
    zc = jax.lax.dot_general(zi, cc, (((1,), (1,)), ((), ())),
                             preferred_element_type=jnp.float32)
    z2 = jnp.sum(zi * zi, axis=1, keepdims=True)
    c2 = jnp.sum(cc * cc, axis=1)[None, :]
    d2 = z2 - 2.0 * zc + c2
    qu = 1.0 / (1.0 + d2)
    q_ref[...] = qu / jnp.sum(qu, axis=1, keepdims=True)


def _dense_outputs(z, cluster_centers):
    return pl.pallas_call(
        _dense_body,
        grid=(N // ROW_BLK,),
        in_specs=[
            pl.BlockSpec((ROW_BLK, D_OUT), lambda i: (i, 0)),
            pl.BlockSpec((N, D_OUT), lambda i: (0, 0)),
            pl.BlockSpec((K, D_OUT), lambda i: (0, 0)),
        ],
        out_specs=[
            pl.BlockSpec((ROW_BLK, N), lambda i: (i, 0)),
            pl.BlockSpec((ROW_BLK, K), lambda i: (i, 0)),
        ],
        out_shape=[
            jax.ShapeDtypeStruct((N, N), jnp.float32),
            jax.ShapeDtypeStruct((N, K), jnp.float32),
        ],
    )(z, z, cluster_centers)


def kernel(x, edge_index, W1, a1_src, a1_dst, b1, W2, a2_src, a2_dst, b2,
           cluster_centers):
    src, dst = edge_index[0], edge_index[1]
    loop = jnp.arange(N, dtype=jnp.int32)
    fill = jnp.full((EPAD - E - N,), N, jnp.int32)
    src_flat = jnp.concatenate([src, loop, fill])
    dst_flat = jnp.concatenate([dst, loop, fill])
    src3a = _edge_layout(src_flat, CHUNK1, NCHF1, NCHS1)
    dst3a = _edge_layout(dst_flat, CHUNK1, NCHF1, NCHS1)
    src3b = _edge_layout(src_flat, CHUNK2, NCHF2, NCHS2)
    dst3b = _edge_layout(dst_flat, CHUNK2, NCHF2, NCHS2)

    x_pad = jnp.pad(x, ((0, NPAD - N), (0, 0)))
    a1 = jnp.stack([a1_src, a1_dst])
    a2 = jnp.stack([a2_src, a2_dst])
    zrows1 = jnp.zeros((CHUNK1, DP1), jnp.float32)
    zrows2 = jnp.zeros((CHUNK2, DP2), jnp.float32)

    hp1, adt1 = _pre1(x_pad, W1, a1)
    acc1 = _sc_gat1(hp1, adt1, src3a, dst3a, zrows1)
    hp2, adt2 = _mid(acc1, b1[None, :], W2, a2)
    acc2 = _sc_gat2(hp2, adt2, src3b, dst3b, zrows2)
    z = _fin(acc2, b2[None, :])
    a_pred, q = _dense_outputs(z, cluster_centers)
    return (z, a_pred, q)


# speedup vs baseline: 1.3500x; 1.0545x over previous
import functools

import jax
import jax.numpy as jnp
from jax import lax
from jax.experimental import pallas as pl
from jax.experimental.pallas import tpu as pltpu
from jax.experimental.pallas import tpu_sc as plsc

N = 10000
E = 160000
D_IN = 128
D_HID = 128
D_OUT = 64
K = 16

NC = 2
NS = 16
NW = NC * NS

NPAD = 10240
ROWS_PER_SUB = NPAD // NS
STG = ROWS_PER_SUB // 2

EPAD = 172032
SLOW_CORE = 0
CHUNK1 = 64
NCHF1, NCHS1 = 84, 84
CHUNK2 = 128
NCHF2, NCHS2 = 42, 42

DP1 = 144
DP2 = 80

_mesh = plsc.VectorSubcoreMesh(
    core_axis_name="c", subcore_axis_name="s", num_cores=NC, num_subcores=NS
)


def _make_sc_gat(dp):
    acol = dp - 15

    def make_sc(chunk, nch_fast, nch_slow, slow_core, spmem_table=False):
        assert nch_fast % 3 == 0 and nch_slow % 3 == 0
        peel = 3

        def body(hp, adt, edg3, zrows, out,
                 eidx0, eidx1, eidx2,
                 dsc0, dsc1, dsc2, wc_v, rows0, rows1, rows2,
                 alr0, alr1, alr2,
                 si0, si1, si2, sg0, sg1, sg2, ss0, ss1, ss2, acc_sh,
                 *maybe_tbl):
            tbl = maybe_tbl[0] if spmem_table else hp
            c = lax.axis_index("c")
            s = lax.axis_index("s")
            wid = s * NC + c
            nchunk = jnp.where(c == slow_core, nch_slow, nch_fast)
            last = nchunk - 1
            nloops = nchunk // 3 - 1
            eidx = (eidx0, eidx1, eidx2)
            dsc = (dsc0, dsc1, dsc2)
            rows = (rows0, rows1, rows2)
            alr = (alr0, alr1, alr2)
            sem_i = (si0, si1, si2)
            sem_g = (sg0, sg1, sg2)
            sem_s = (ss0, ss1, ss2)
            sidx = tuple(e.at[0] for e in eidx)
            didx = tuple(e.at[1] for e in eidx)

            def issue_idx(ci, b):
                pltpu.async_copy(edg3.at[wid, ci], eidx[b], sem_i[b])

            def wait_idx(b):
                pltpu.make_async_copy(edg3.at[wid, 0], eidx[b], sem_i[b]).wait()

            def issue_gather(b):
                pltpu.async_copy(tbl.at[sidx[b]], rows[b], sem_g[b])
                pltpu.async_copy(adt.at[didx[b]], alr[b], sem_g[b])

            def wait_gather(b):
                pltpu.make_async_copy(tbl.at[sidx[b]], rows[b], sem_g[b]).wait()
                pltpu.make_async_copy(adt.at[didx[b]], alr[b], sem_g[b]).wait()

            def wait_scat(b):
                pltpu.make_async_copy(rows[b], acc_sh.at[dsc[b]],
                                      sem_s[b]).wait()

            issue_idx(0, 0)
            issue_idx(1, 1)
            issue_idx(2, 2)

            base = s * ROWS_PER_SUB
            nzc = ROWS_PER_SUB // chunk
            if spmem_table:
                for k in range(nzc):
                    sl = pl.ds(base + k * chunk, chunk)
                    pltpu.sync_copy(hp.at[sl], rows0)
                    pltpu.sync_copy(rows0, tbl.at[sl])
            pltpu.sync_copy(zrows, rows0)
            for k in range(nzc):
                pltpu.sync_copy(rows0, acc_sh.at[pl.ds(base + k * chunk, chunk)])
            rem = ROWS_PER_SUB - nzc * chunk
            if rem:
                pltpu.sync_copy(rows0.at[pl.ds(0, rem)],
                                acc_sh.at[pl.ds(base + nzc * chunk, rem)])
            plsc.subcore_barrier()

            wait_idx(0)
            issue_gather(0)

            iota16 = lax.iota(jnp.int32, 16)
            zeros16 = jnp.zeros((16,), jnp.int32)
            acol16 = jnp.full((16,), acol, jnp.int32)

            def phase(ci, b, first):
                p = (b + 1) % 3
                if not first:
                    wait_scat(p)
                wait_idx(p)
                issue_gather(p)
                wait_gather(b)
                for j in range(chunk // 16):
                    sl = pl.ds(j * 16, 16)
                    dsc[b][sl] = eidx[b][1, sl]
                issue_idx(jnp.minimum(ci + 3, last), b)
                for j in range(chunk // 16):
                    lane = j * 16 + iota16
                    av = plsc.load_gather(rows[b], [lane, acol16])
                    dv = plsc.load_gather(alr[b], [lane, zeros16])
                    e = av + dv
                    e = jnp.where(e >= 0.0, e, 0.2 * e)
                    wc_v[pl.ds(j * 16, 16)] = jnp.exp(e)
                def scale_step(ei, _):
                    wsp = plsc.load_gather(
                        wc_v, [jnp.full((16,), ei, jnp.int32)])
                    for j in range(dp // 16):
                        sl = pl.ds(j * 16, 16)
                        rows[b][ei, sl] = rows[b][ei, sl] * wsp
                    return 0
                lax.fori_loop(0, chunk, scale_step, 0, unroll=4)
                pltpu.async_copy(rows[b], acc_sh.at[dsc[b]], sem_s[b],
                                 add=True)

            for ci in range(peel):
                phase(ci, ci % 3, ci < 2)

            def loop_body(i3, _):
                ci = peel + 3 * i3
                phase(ci, peel % 3, False)
                phase(ci + 1, (peel + 1) % 3, False)
                phase(ci + 2, (peel + 2) % 3, False)
                return 0

            lax.fori_loop(0, nloops, loop_body, 0)

            wait_scat(1)
            wait_scat(2)
            wait_idx(1)
            wait_idx(2)
            wait_gather(0)
            plsc.subcore_barrier()

            for k in range(nzc):
                sl = pl.ds(base + k * chunk, chunk)
                pltpu.sync_copy(acc_sh.at[sl], rows0)
                pltpu.sync_copy(rows0, out.at[c, sl])
            if rem:
                sl = pl.ds(base + nzc * chunk, rem)
                pltpu.sync_copy(acc_sh.at[sl], rows0.at[pl.ds(0, rem)])
                pltpu.sync_copy(rows0.at[pl.ds(0, rem)], out.at[c, sl])

        return pl.kernel(
            body,
            out_type=jax.ShapeDtypeStruct((NC, NPAD, dp), jnp.float32),
            mesh=_mesh,
            compiler_params=pltpu.CompilerParams(
                needs_layout_passes=False, use_tc_tiling_on_sc=False),
            scratch_types=(
                [pltpu.VMEM((2, chunk), jnp.int32)] * 3 +
                [pltpu.VMEM((chunk,), jnp.int32)] * 3 +
                [pltpu.VMEM((chunk,), jnp.float32)] +
                [pltpu.VMEM((chunk, dp), jnp.float32)] * 3 +
                [pltpu.VMEM((chunk, 16), jnp.float32)] * 3 +
                [pltpu.SemaphoreType.DMA] * 9 +
                [pltpu.VMEM_SHARED((NPAD, dp), jnp.float32)] +
                ([pltpu.VMEM_SHARED((NPAD, dp), jnp.float32)]
                 if spmem_table else [])
            ),
        )

    return make_sc


_sc_gat1 = _make_sc_gat(DP1)(CHUNK1, NCHF1, NCHS1, SLOW_CORE)
_sc_gat2 = _make_sc_gat(DP2)(CHUNK2, NCHF2, NCHS2, SLOW_CORE)


def _edge_layout(flat, chunk, nf, ns):
    if nf == ns:
        return flat.reshape(NW, nf, chunk)
    lf, ls = nf * chunk, ns * chunk
    seg_f = flat[:NS * lf].reshape(NS, lf)
    seg_s = flat[NS * lf:NS * (lf + ls)].reshape(NS, ls)
    arr = jnp.full((NW, nf * chunk), N, jnp.int32)
    arr = arr.at[(1 - SLOW_CORE)::2, :lf].set(seg_f)
    arr = arr.at[SLOW_CORE::2, :ls].set(seg_s)
    return arr.reshape(NW, nf, chunk)


BLK_A = 1280


def _tail_cols(blk, al, nblk):
    rows = pl.program_id(0) * nblk + lax.broadcasted_iota(
        jnp.int32, (nblk, 16), 0)
    lanes = lax.broadcasted_iota(jnp.int32, (nblk, 16), 1)
    ones = jnp.where((lanes == 0) & (rows < N), 1.0, 0.0)
    return ones + jnp.where(lanes == 1, al, 0.0)


def _pre1_body(x_ref, w_ref, a_ref, hp_ref, adt_ref):
    xb = x_ref[...]
    h = jax.lax.dot_general(xb, w_ref[...], (((1,), (0,)), ((), ())),
                            preferred_element_type=jnp.float32)
    al = jax.lax.dot_general(h, a_ref[...], (((1,), (1,)), ((), ())),
                             preferred_element_type=jnp.float32)
    hp_ref[:, :D_HID] = h
    hp_ref[:, D_HID:DP1] = _tail_cols(BLK_A, al[:, 0:1], BLK_A)
    lanes = lax.broadcasted_iota(jnp.int32, (BLK_A, 16), 1)
    adt_ref[...] = jnp.where(lanes == 0, al[:, 1:2], 0.0)


def _pre1(x_pad, w1, a1):
    return pl.pallas_call(
        _pre1_body,
        grid=(NPAD // BLK_A,),
        in_specs=[
            pl.BlockSpec((BLK_A, D_IN), lambda i: (i, 0)),
            pl.BlockSpec((D_IN, D_HID), lambda i: (0, 0)),
            pl.BlockSpec((2, D_HID), lambda i: (0, 0)),
        ],
        out_specs=[
            pl.BlockSpec((BLK_A, DP1), lambda i: (i, 0)),
            pl.BlockSpec((BLK_A, 16), lambda i: (i, 0)),
        ],
        out_shape=[
            jax.ShapeDtypeStruct((NPAD, DP1), jnp.float32),
            jax.ShapeDtypeStruct((NPAD, 16), jnp.float32),
        ],
    )(x_pad, w1, a1)


BLK_B = 1280


def _mid_body(acc_ref, b1_ref, w2_ref, a2_ref, hp2_ref, adt2_ref):
    accs = acc_ref[0] + acc_ref[1]
    num = accs[:, :D_HID]
    den = accs[:, D_HID:D_HID + 1]
    h1 = num / jnp.maximum(den, 1e-30) + b1_ref[...]
    h1 = jnp.where(h1 > 0.0, h1, jnp.exp(jnp.minimum(h1, 0.0)) - 1.0)
    h2 = jax.lax.dot_general(h1, w2_ref[...], (((1,), (0,)), ((), ())),
                             preferred_element_type=jnp.float32)
    al = jax.lax.dot_general(h2, a2_ref[...], (((1,), (1,)), ((), ())),
                             preferred_element_type=jnp.float32)
    hp2_ref[:, :D_OUT] = h2
    hp2_ref[:, D_OUT:DP2] = _tail_cols(BLK_B, al[:, 0:1], BLK_B)
    lanes = lax.broadcasted_iota(jnp.int32, (BLK_B, 16), 1)
    adt2_ref[...] = jnp.where(lanes == 0, al[:, 1:2], 0.0)


def _mid(acc1, b1, w2, a2):
    return pl.pallas_call(
        _mid_body,
        grid=(NPAD // BLK_B,),
        in_specs=[
            pl.BlockSpec((NC, BLK_B, DP1), lambda i: (0, i, 0)),
            pl.BlockSpec((1, D_HID), lambda i: (0, 0)),
            pl.BlockSpec((D_HID, D_OUT), lambda i: (0, 0)),
            pl.BlockSpec((2, D_OUT), lambda i: (0, 0)),
        ],
        out_specs=[
            pl.BlockSpec((BLK_B, DP2), lambda i: (i, 0)),
            pl.BlockSpec((BLK_B, 16), lambda i: (i, 0)),
        ],
        out_shape=[
            jax.ShapeDtypeStruct((NPAD, DP2), jnp.float32),
            jax.ShapeDtypeStruct((NPAD, 16), jnp.float32),
        ],
    )(acc1, b1, w2, a2)


BLK_C = 2000


def _fin_body(acc_ref, b2_ref, z_ref):
    accs = acc_ref[0] + acc_ref[1]
    num = accs[:, :D_OUT]
    den = accs[:, D_OUT:D_OUT + 1]
    z_ref[...] = num / jnp.maximum(den, 1e-30) + b2_ref[...]


def _fin(acc2, b2):
    return pl.pallas_call(
        _fin_body,
        grid=(N // BLK_C,),
        in_specs=[
            pl.BlockSpec((NC, BLK_C, DP2), lambda i: (0, i, 0)),
            pl.BlockSpec((1, D_OUT), lambda i: (0, 0)),
        ],
        out_specs=pl.BlockSpec((BLK_C, D_OUT), lambda i: (i, 0)),
        out_shape=jax.ShapeDtypeStruct((N, D_OUT), jnp.float32),
    )(acc2, b2)


ROW_BLK = 400


def _dense_body(z_blk_ref, z_all_ref, cc_ref, a_ref, q_ref):
    zi = z_blk_ref[...]
    zall = z_all_ref[...]
    cc = cc_ref[...]
    sim = jax.lax.dot_general(zi, zall, (((1,), (1,)), ((), ())),
                              preferred_element_type=jnp.float32)
    a_ref[...] = jax.nn.sigmoid(sim)
    zc = jax.lax.dot_general(zi, cc, (((1,), (1,)), ((), ())),
                             preferred_element_type=jnp.float32)
    z2 = jnp.sum(zi * zi, axis=1, keepdims=True)
    c2 = jnp.sum(cc * cc, axis=1)[None, :]
    d2 = z2 - 2.0 * zc + c2
    qu = 1.0 / (1.0 + d2)
    q_ref[...] = qu / jnp.sum(qu, axis=1, keepdims=True)


def _dense_outputs(z, cluster_centers):
    return pl.pallas_call(
        _dense_body,
        grid=(N // ROW_BLK,),
        in_specs=[
            pl.BlockSpec((ROW_BLK, D_OUT), lambda i: (i, 0)),
            pl.BlockSpec((N, D_OUT), lambda i: (0, 0)),
            pl.BlockSpec((K, D_OUT), lambda i: (0, 0)),
        ],
        out_specs=[
            pl.BlockSpec((ROW_BLK, N), lambda i: (i, 0)),
            pl.BlockSpec((ROW_BLK, K), lambda i: (i, 0)),
        ],
        out_shape=[
            jax.ShapeDtypeStruct((N, N), jnp.float32),
            jax.ShapeDtypeStruct((N, K), jnp.float32),
        ],
    )(z, z, cluster_centers)


def kernel(x, edge_index, W1, a1_src, a1_dst, b1, W2, a2_src, a2_dst, b2,
           cluster_centers):
    src, dst = edge_index[0], edge_index[1]
    loop = jnp.arange(N, dtype=jnp.int32)
    fill = jnp.full((EPAD - E - N,), N, jnp.int32)
    src_flat = jnp.concatenate([src, loop, fill])
    dst_flat = jnp.concatenate([dst, loop, fill])
    edg3a = jnp.stack([_edge_layout(src_flat, CHUNK1, NCHF1, NCHS1),
                       _edge_layout(dst_flat, CHUNK1, NCHF1, NCHS1)], axis=2)
    edg3b = jnp.stack([_edge_layout(src_flat, CHUNK2, NCHF2, NCHS2),
                       _edge_layout(dst_flat, CHUNK2, NCHF2, NCHS2)], axis=2)

    x_pad = jnp.pad(x, ((0, NPAD - N), (0, 0)))
    a1 = jnp.stack([a1_src, a1_dst])
    a2 = jnp.stack([a2_src, a2_dst])
    zrows1 = jnp.zeros((CHUNK1, DP1), jnp.float32)
    zrows2 = jnp.zeros((CHUNK2, DP2), jnp.float32)

    hp1, adt1 = _pre1(x_pad, W1, a1)
    acc1 = _sc_gat1(hp1, adt1, edg3a, zrows1)
    hp2, adt2 = _mid(acc1, b1[None, :], W2, a2)
    acc2 = _sc_gat2(hp2, adt2, edg3b, zrows2)
    z = _fin(acc2, b2[None, :])
    a_pred, q = _dense_outputs(z, cluster_centers)
    return (z, a_pred, q)
